# Initial kernel scaffold; baseline (speedup 1.0000x reference)
import functools

import jax
import jax.numpy as jnp
from jax import lax
from jax.experimental import pallas as pl
from jax.experimental.pallas import tpu as pltpu
from jax.experimental.pallas import tpu_sc as plsc

N = 10000
D = 256
H = 512
E = 160000
HN = N // 2
PR = 5008
PADN = 2 * PR
TRASH = HN
NC = 2
NS = 16
EC = 128
CPT = 79
EPT = CPT * EC
EPAD = NS * EPT
EROWS = EPAD // EC
RPT = PR // NS
DGR = 5120
DGW = 16
BLK = 2504

_mesh = plsc.VectorSubcoreMesh(core_axis_name="c", subcore_axis_name="s")


def _deg_body(dst_hbm, deg_hbm, dl_v, ones_v, zer_v, deg_sh):
    c = lax.axis_index("c")
    s = lax.axis_index("s")
    pltpu.sync_copy(dst_hbm.at[pl.ds(s * CPT, CPT)], dl_v)

    base = c * HN

    def tr(i, _):
        j = i // 8
        kk = (i % 8) * 16
        d = dl_v[j, pl.ds(kk, 16)]
        dloc = d - base
        m = (dloc >= 0) & (dloc < HN)
        dl_v[j, pl.ds(kk, 16)] = jnp.where(m, dloc, TRASH)
        return 0

    lax.fori_loop(0, CPT * 8, tr, 0)

    for i in range(8 * DGW):
        ones_v[pl.ds(i * 16, 16)] = jnp.full((16,), 1.0, jnp.float32)
    for i in range(20 * DGW):
        zer_v[pl.ds(i * 16, 16)] = jnp.zeros((16,), jnp.float32)
    ones2 = ones_v.reshape(EC, DGW)
    zer2 = zer_v.reshape(DGR // NS, DGW)
    pltpu.sync_copy(zer2, deg_sh.at[pl.ds(s * (DGR // NS), DGR // NS)])
    plsc.subcore_barrier()

    def sc(k, _):
        pltpu.sync_copy(ones2, deg_sh.at[dl_v.at[k]], add=True)
        return 0

    lax.fori_loop(0, CPT, sc, 0)
    plsc.subcore_barrier()
    pltpu.sync_copy(deg_sh.at[pl.ds(s * (DGR // NS), DGR // NS)],
                    deg_hbm.at[c, pl.ds(s * (DGR // NS), DGR // NS)])


_deg_call = functools.partial(
    pl.kernel,
    out_type=jax.ShapeDtypeStruct((NC, DGR, DGW), jnp.float32),
    mesh=_mesh,
    scratch_types=[
        pltpu.VMEM((CPT, EC), jnp.int32),
        pltpu.VMEM((EC * DGW,), jnp.float32),
        pltpu.VMEM(((DGR // NS) * DGW,), jnp.float32),
        pltpu.VMEM_SHARED((DGR, DGW), jnp.float32),
    ],
)(_deg_body)


def _agg_body(y_hbm, src_hbm, dst_hbm, out_hbm,
              sr_v, dr_v, cs1_v, cd1_v, cs2_v, cd2_v, rows_v, acc_sh):
    c = lax.axis_index("c")
    s = lax.axis_index("s")
    pltpu.sync_copy(src_hbm.at[pl.ds(s * CPT, CPT)], sr_v)
    pltpu.sync_copy(dst_hbm.at[pl.ds(s * CPT, CPT)], dr_v)

    base = c * HN

    def cp(i, off):
        j = i // 8
        kk = (i % 8) * 16
        sv = sr_v[j, pl.ds(kk, 16)]
        dv = dr_v[j, pl.ds(kk, 16)]
        dloc = dv - base
        m = (dloc >= 0) & (dloc < HN)
        sp = sv + jnp.where(sv >= HN, 8, 0)
        plsc.store_compressed(cs1_v.at[pl.ds(off, 16)], sp, m)
        plsc.store_compressed(cd1_v.at[pl.ds(off, 16)], dloc, m)
        return off + jnp.sum(m.astype(jnp.int32))

    ncomp = lax.fori_loop(0, CPT * 8, cp, jnp.int32(0))

    def padl(i, _):
        cs1_v[pl.ds(ncomp + i * 16, 16)] = jnp.zeros((16,), jnp.int32)
        cd1_v[pl.ds(ncomp + i * 16, 16)] = jnp.full((16,), TRASH, jnp.int32)
        return 0

    lax.fori_loop(0, 8, padl, 0)
    nch = (ncomp + EC - 1) // EC

    def c2(i, _):
        cs2_v[i // 8, pl.ds((i % 8) * 16, 16)] = cs1_v[pl.ds(i * 16, 16)]
        cd2_v[i // 8, pl.ds((i % 8) * 16, 16)] = cd1_v[pl.ds(i * 16, 16)]
        return 0

    lax.fori_loop(0, nch * 8, c2, 0)

    pltpu.sync_copy(y_hbm.at[pl.ds(c * PR + s * RPT, RPT)],
                    acc_sh.at[pl.ds(s * RPT, RPT)])
    plsc.subcore_barrier()

    def gs(k, _):
        pltpu.sync_copy(y_hbm.at[cs2_v.at[k]], rows_v)
        pltpu.sync_copy(rows_v, acc_sh.at[cd2_v.at[k]], add=True)
        return 0

    lax.fori_loop(0, nch, gs, 0)
    plsc.subcore_barrier()
    pltpu.sync_copy(acc_sh.at[pl.ds(s * RPT, RPT)],
                    out_hbm.at[c, pl.ds(s * RPT, RPT)])


_agg_call = functools.partial(
    pl.kernel,
    out_type=jax.ShapeDtypeStruct((NC, PR, D), jnp.float32),
    mesh=_mesh,
    scratch_types=[
        pltpu.VMEM((CPT, EC), jnp.int32),
        pltpu.VMEM((CPT, EC), jnp.int32),
        pltpu.VMEM((EPT + EC,), jnp.int32),
        pltpu.VMEM((EPT + EC,), jnp.int32),
        pltpu.VMEM((CPT, EC), jnp.int32),
        pltpu.VMEM((CPT, EC), jnp.int32),
        pltpu.VMEM((EC, D), jnp.float32),
        pltpu.VMEM_SHARED((PR, D), jnp.float32),
    ],
)(_agg_body)


def _scale_body(x_ref, deg_ref, o_ref):
    dis = lax.rsqrt(deg_ref[...] + 1.0)
    o_ref[...] = x_ref[...] * dis


_scale_call = pl.pallas_call(
    _scale_body,
    grid=(PADN // BLK,),
    in_specs=[
        pl.BlockSpec((BLK, D), lambda i: (i, 0)),
        pl.BlockSpec((BLK, 1), lambda i: (i, 0)),
    ],
    out_specs=pl.BlockSpec((BLK, D), lambda i: (i, 0)),
    out_shape=jax.ShapeDtypeStruct((PADN, D), jnp.float32),
)


def _mid_body(s1_ref, deg_ref, w1_ref, b1_ref, w2_ref, y2_ref):
    dis = lax.rsqrt(deg_ref[...] + 1.0)
    agg = s1_ref[...] * dis
    h1 = jnp.maximum(
        jnp.dot(agg, w1_ref[...], preferred_element_type=jnp.float32)
        + b1_ref[...], 0.0)
    t = jnp.dot(h1, w2_ref[...], preferred_element_type=jnp.float32)
    y2_ref[...] = t * dis


_mid_call = pl.pallas_call(
    _mid_body,
    grid=(PADN // BLK,),
    in_specs=[
        pl.BlockSpec((BLK, D), lambda i: (i, 0)),
        pl.BlockSpec((BLK, 1), lambda i: (i, 0)),
        pl.BlockSpec((D, H), lambda i: (0, 0)),
        pl.BlockSpec((1, H), lambda i: (0, 0)),
        pl.BlockSpec((H, D), lambda i: (0, 0)),
    ],
    out_specs=pl.BlockSpec((BLK, D), lambda i: (i, 0)),
    out_shape=jax.ShapeDtypeStruct((PADN, D), jnp.float32),
)


def _dec_body(s2_ref, deg_ref, b2_ref, wd1_ref, bd1_ref, wd2_ref, bd2_ref,
              xh_ref, z_ref):
    dis = lax.rsqrt(deg_ref[...] + 1.0)
    z = s2_ref[...] * dis + b2_ref[...]
    z_ref[...] = z
    h = jnp.maximum(
        jnp.dot(z, wd1_ref[...], preferred_element_type=jnp.float32)
        + bd1_ref[...], 0.0)
    xh_ref[...] = (jnp.dot(h, wd2_ref[...], preferred_element_type=jnp.float32)
                   + bd2_ref[...])


_dec_call = pl.pallas_call(
    _dec_body,
    grid=(PADN // BLK,),
    in_specs=[
        pl.BlockSpec((BLK, D), lambda i: (i, 0)),
        pl.BlockSpec((BLK, 1), lambda i: (i, 0)),
        pl.BlockSpec((1, D), lambda i: (0, 0)),
        pl.BlockSpec((D, H), lambda i: (0, 0)),
        pl.BlockSpec((1, H), lambda i: (0, 0)),
        pl.BlockSpec((H, D), lambda i: (0, 0)),
        pl.BlockSpec((1, D), lambda i: (0, 0)),
    ],
    out_specs=[
        pl.BlockSpec((BLK, D), lambda i: (i, 0)),
        pl.BlockSpec((BLK, D), lambda i: (i, 0)),
    ],
    out_shape=[
        jax.ShapeDtypeStruct((PADN, D), jnp.float32),
        jax.ShapeDtypeStruct((PADN, D), jnp.float32),
    ],
)


def kernel(x, edge_index, W1, b1, W2, b2, Wd1, bd1, Wd2, bd2):
    ei = edge_index.astype(jnp.int32)
    src = jnp.concatenate(
        [ei[0], jnp.zeros((EPAD - E,), jnp.int32)]).reshape(EROWS, EC)
    dst = jnp.concatenate(
        [ei[1], jnp.full((EPAD - E,), 1 << 30, jnp.int32)]).reshape(EROWS, EC)

    dego = _deg_call(dst)
    deg = jnp.concatenate([dego[0, :PR, 0], dego[1, :PR, 0]]).reshape(PADN, 1)

    zpad = jnp.zeros((PR - HN, D), jnp.float32)
    x_p = jnp.concatenate([x[:HN], zpad, x[HN:], zpad], axis=0)

    y1 = _scale_call(x_p, deg)
    s1 = _agg_call(y1, src, dst).reshape(PADN, D)
    y2 = _mid_call(s1, deg, W1, b1.reshape(1, H), W2)
    s2 = _agg_call(y2, src, dst).reshape(PADN, D)
    xh_p, z_p = _dec_call(s2, deg, b2.reshape(1, D), Wd1,
                          bd1.reshape(1, H), Wd2, bd2.reshape(1, D))

    z = jnp.concatenate([z_p[:HN], z_p[PR:PR + HN]], axis=0)
    x_hat = jnp.concatenate([xh_p[:HN], xh_p[PR:PR + HN]], axis=0)
    return (x_hat, z)


# speedup vs baseline: 2.7280x; 2.7280x over previous
import functools

import jax
import jax.numpy as jnp
from jax import lax
from jax.experimental import pallas as pl
from jax.experimental.pallas import tpu as pltpu
from jax.experimental.pallas import tpu_sc as plsc

N = 10000
D = 256
DH = 128
H = 512
E = 160000
NC = 2
NS = 16
EC = 128
PADN = 10240
EPAD = 163840
AEPW = EPAD // NS
ACPT = AEPW // EC
DEPW = EPAD // (NC * NS)
DCPT = DEPW // EC
RPT = PADN // NS
DGW = 16
BLK = 2560

_mesh = plsc.VectorSubcoreMesh(core_axis_name="c", subcore_axis_name="s")


def _deg_body(dstf_hbm, ones_hbm, zer_hbm, deg_hbm, di_v, ones_v, deg_sh):
    c = lax.axis_index("c")
    s = lax.axis_index("s")
    pltpu.sync_copy(ones_hbm, ones_v)
    pltpu.sync_copy(zer_hbm.at[pl.ds(s * RPT, RPT)],
                    deg_sh.at[pl.ds(s * RPT, RPT)])
    plsc.subcore_barrier()

    base = (c * NS + s) * DEPW

    def sc(k, _):
        pltpu.sync_copy(dstf_hbm.at[pl.ds(base + k * EC, EC)], di_v)
        pltpu.sync_copy(ones_v, deg_sh.at[di_v], add=True)
        return 0

    lax.fori_loop(0, DCPT, sc, 0)
    plsc.subcore_barrier()
    pltpu.sync_copy(deg_sh.at[pl.ds(s * RPT, RPT)],
                    deg_hbm.at[c, pl.ds(s * RPT, RPT)])


_deg_call = functools.partial(
    pl.kernel,
    out_type=jax.ShapeDtypeStruct((NC, PADN, DGW), jnp.float32),
    mesh=_mesh,
    scratch_types=[
        pltpu.VMEM((EC,), jnp.int32),
        pltpu.VMEM((EC, DGW), jnp.float32),
        pltpu.VMEM_SHARED((PADN, DGW), jnp.float32),
    ],
)(_deg_body)


def _dis(deg_ref):
    return lax.rsqrt(deg_ref[0][:, :1] + deg_ref[1][:, :1] + 1.0)


def _scale_body(x_ref, deg_ref, o_ref):
    y = x_ref[...] * _dis(deg_ref)
    o_ref[0] = y[:, :DH]
    o_ref[1] = y[:, DH:]


_scale_call = pl.pallas_call(
    _scale_body,
    grid=(PADN // BLK,),
    in_specs=[
        pl.BlockSpec((BLK, D), lambda i: (i, 0)),
        pl.BlockSpec((NC, BLK, DGW), lambda i: (0, i, 0)),
    ],
    out_specs=pl.BlockSpec((NC, BLK, DH), lambda i: (0, i, 0)),
    out_shape=jax.ShapeDtypeStruct((NC, PADN, DH), jnp.float32),
)


def _mid_body(s1_ref, deg_ref, w1_ref, b1_ref, w2_ref, y2_ref):
    dis = _dis(deg_ref)
    agg = jnp.concatenate([s1_ref[0], s1_ref[1]], axis=1) * dis
    h1 = jnp.maximum(
        jnp.dot(agg, w1_ref[...], preferred_element_type=jnp.float32)
        + b1_ref[...], 0.0)
    t = jnp.dot(h1, w2_ref[...], preferred_element_type=jnp.float32) * dis
    y2_ref[0] = t[:, :DH]
    y2_ref[1] = t[:, DH:]


_mid_call = pl.pallas_call(
    _mid_body,
    grid=(PADN // BLK,),
    in_specs=[
        pl.BlockSpec((NC, BLK, DH), lambda i: (0, i, 0)),
        pl.BlockSpec((NC, BLK, DGW), lambda i: (0, i, 0)),
        pl.BlockSpec((D, H), lambda i: (0, 0)),
        pl.BlockSpec((1, H), lambda i: (0, 0)),
        pl.BlockSpec((H, D), lambda i: (0, 0)),
    ],
    out_specs=pl.BlockSpec((NC, BLK, DH), lambda i: (0, i, 0)),
    out_shape=jax.ShapeDtypeStruct((NC, PADN, DH), jnp.float32),
)


def _dec_body(s2_ref, deg_ref, b2_ref, wd1_ref, bd1_ref, wd2_ref, bd2_ref,
              xh_ref, z_ref):
    dis = _dis(deg_ref)
    z = jnp.concatenate([s2_ref[0], s2_ref[1]], axis=1) * dis + b2_ref[...]
    z_ref[...] = z
    h = jnp.maximum(
        jnp.dot(z, wd1_ref[...], preferred_element_type=jnp.float32)
        + bd1_ref[...], 0.0)
    xh_ref[...] = (jnp.dot(h, wd2_ref[...], preferred_element_type=jnp.float32)
                   + bd2_ref[...])


_dec_call = pl.pallas_call(
    _dec_body,
    grid=(PADN // BLK,),
    in_specs=[
        pl.BlockSpec((NC, BLK, DH), lambda i: (0, i, 0)),
        pl.BlockSpec((NC, BLK, DGW), lambda i: (0, i, 0)),
        pl.BlockSpec((1, D), lambda i: (0, 0)),
        pl.BlockSpec((D, H), lambda i: (0, 0)),
        pl.BlockSpec((1, H), lambda i: (0, 0)),
        pl.BlockSpec((H, D), lambda i: (0, 0)),
        pl.BlockSpec((1, D), lambda i: (0, 0)),
    ],
    out_specs=[
        pl.BlockSpec((BLK, D), lambda i: (i, 0)),
        pl.BlockSpec((BLK, D), lambda i: (i, 0)),
    ],
    out_shape=[
        jax.ShapeDtypeStruct((PADN, D), jnp.float32),
        jax.ShapeDtypeStruct((PADN, D), jnp.float32),
    ],
)


def _sagg_body(y_hbm, msgs_hbm, dstf_hbm, out_hbm, di_v, rows_v, acc_sh):
    c = lax.axis_index("c")
    s = lax.axis_index("s")
    pltpu.sync_copy(y_hbm.at[pl.ds(c * PADN + s * RPT, RPT)],
                    acc_sh.at[pl.ds(s * RPT, RPT)])
    plsc.subcore_barrier()

    base = s * AEPW

    def gs(k, _):
        off = base + k * EC
        pltpu.sync_copy(dstf_hbm.at[pl.ds(off, EC)], di_v)
        pltpu.sync_copy(msgs_hbm.at[c, pl.ds(off, EC)], rows_v)
        pltpu.sync_copy(rows_v, acc_sh.at[di_v], add=True)
        return 0

    lax.fori_loop(0, ACPT, gs, 0)
    plsc.subcore_barrier()
    pltpu.sync_copy(acc_sh.at[pl.ds(s * RPT, RPT)],
                    out_hbm.at[c, pl.ds(s * RPT, RPT)])


_sagg_call = functools.partial(
    pl.kernel,
    out_type=jax.ShapeDtypeStruct((NC, PADN, DH), jnp.float32),
    mesh=_mesh,
    scratch_types=[
        pltpu.VMEM((EC,), jnp.int32),
        pltpu.VMEM((EC, DH), jnp.float32),
        pltpu.VMEM_SHARED((PADN, DH), jnp.float32),
    ],
)(_sagg_body)


def kernel(x, edge_index, W1, b1, W2, b2, Wd1, bd1, Wd2, bd2):
    ei = edge_index.astype(jnp.int32)
    src = jnp.concatenate([ei[0], jnp.zeros((EPAD - E,), jnp.int32)])
    dst = jnp.concatenate([ei[1], jnp.full((EPAD - E,), N, jnp.int32)])
    src2 = jnp.stack([src, src + PADN])
    ones = jnp.ones((EC, DGW), jnp.float32)
    zer = jnp.zeros((PADN, DGW), jnp.float32)

    dego = _deg_call(dst, ones, zer)

    x_p = jnp.concatenate([x, jnp.zeros((PADN - N, D), jnp.float32)], axis=0)
    y1 = _scale_call(x_p, dego)
    yf1 = y1.reshape(NC * PADN, DH)
    s1 = _sagg_call(yf1, yf1[src2], dst)
    y2 = _mid_call(s1, dego, W1, b1.reshape(1, H), W2)
    yf2 = y2.reshape(NC * PADN, DH)
    s2 = _sagg_call(yf2, yf2[src2], dst)
    xh_p, z_p = _dec_call(s2, dego, b2.reshape(1, D), Wd1,
                          bd1.reshape(1, H), Wd2, bd2.reshape(1, D))
    return (xh_p[:N], z_p[:N])
